# baseline (speedup 1.0000x reference)
import functools

import jax
import jax.numpy as jnp
from jax import lax
from jax.experimental import pallas as pl
from jax.experimental.pallas import tpu as pltpu
from jax.experimental.pallas import tpu_sc as plsc

N = 10000
E = 320000
D = 128
H = 128
C = 16
NP1 = 1000
E1 = 16000
EPS = 1e-5

NC = 2
NS = 16
NW = NC * NS

POOL_PAD = 10240
E1_PAD = 16384
ARM = 10240
AR1 = 1024
G = 8


def _zdiv(n, cap):
    for z in range(min(n, cap), 0, -1):
        if n % z == 0:
            return z
    return 1


NBUF = 6
LOOKA = 3


def _make_rowsum(EP, CH, AR, GG, NB=NBUF, LA=LOOKA):
    EPT = EP // NW
    NCH = EPT // CH
    NG = NCH // GG
    ART = AR // NS
    ZR = _zdiv(ART, 32)
    NZ = ART // ZR
    mesh = plsc.VectorSubcoreMesh(core_axis_name="c", subcore_axis_name="s")

    @functools.partial(
        pl.kernel,
        out_type=jax.ShapeDtypeStruct((NC * AR, 128), jnp.float32),
        mesh=mesh,
        scratch_types=[
            pltpu.VMEM((GG, CH), jnp.int32),
            pltpu.VMEM((GG, CH), jnp.int32),
            pltpu.VMEM((NB, CH, 128), jnp.float32),
            pltpu.VMEM((ZR, 128), jnp.float32),
            pltpu.VMEM_SHARED((AR, 128), jnp.float32),
        ] + [pltpu.SemaphoreType.DMA] * (2 * NB),
        compiler_params=pltpu.CompilerParams(use_tc_tiling_on_sc=False),
    )
    def k(src_hbm, dst_hbm, table_hbm, z_hbm, out_hbm, idx_s, idx_d,
          rows, zb, acc, *sems):
        gsem = sems[:NB]
        ssem = sems[NB:]
        c = lax.axis_index("c")
        s = lax.axis_index("s")
        w = c * NS + s
        pltpu.sync_copy(z_hbm, zb)

        def zcopy(i, carry):
            pltpu.sync_copy(zb, acc.at[pl.ds(s * ART + i * ZR, ZR)])
            return carry
        lax.fori_loop(0, NZ, zcopy, 0)
        plsc.subcore_barrier()

        def group(gi, carry):
            base = w * NCH + gi * GG
            pltpu.sync_copy(src_hbm.at[pl.ds(base, GG)], idx_s)
            pltpu.sync_copy(dst_hbm.at[pl.ds(base, GG)], idx_d)

            gh = [None] * NB
            sh = [None] * NB
            for j in range(LA):
                gh[j] = pltpu.async_copy(
                    table_hbm.at[idx_s.at[j]], rows.at[j], gsem[j])
            for j in range(GG):
                b = j % NB
                gh[b].wait()
                sh[b] = pltpu.async_copy(
                    rows.at[b], acc.at[idx_d.at[j]], ssem[b], add=True)
                jn = j + LA
                if jn < GG:
                    bn = jn % NB
                    if sh[bn] is not None:
                        sh[bn].wait()
                        sh[bn] = None
                    gh[bn] = pltpu.async_copy(
                        table_hbm.at[idx_s.at[jn]], rows.at[bn], gsem[bn])
            for b in range(NB):
                if sh[b] is not None:
                    sh[b].wait()
            return carry
        lax.fori_loop(0, NG, group, 0)
        plsc.subcore_barrier()

        pltpu.sync_copy(acc.at[pl.ds(s * ART, ART)],
                        out_hbm.at[pl.ds(c * AR + s * ART, ART)])

    def run(src2d, dst2d, table):
        return k(src2d, dst2d, table, jnp.zeros((ZR, 128), jnp.float32))
    return run


def _make_hist_all(CH0, CH1, CH2):
    NCH0 = E // NW // CH0
    NG0 = NCH0 // G
    NCH1 = POOL_PAD // NW // CH1
    NCH2 = E1_PAD // NW // CH2
    ART0 = ARM // NS
    ART1 = AR1 // NS
    ZR = 64
    mesh = plsc.VectorSubcoreMesh(core_axis_name="c", subcore_axis_name="s")
    S16 = lambda AR: jax.ShapeDtypeStruct((NC * AR, 16), jnp.float32)

    @functools.partial(
        pl.kernel,
        out_type=(S16(ARM), S16(AR1), S16(AR1)),
        mesh=mesh,
        scratch_types=[
            pltpu.VMEM((NCH0, CH0), jnp.int32),
            pltpu.VMEM((NCH1, CH1), jnp.int32),
            pltpu.VMEM((NCH2, CH2), jnp.int32),
            pltpu.VMEM((128, 16), jnp.float32),
            pltpu.VMEM((ZR, 16), jnp.float32),
            pltpu.VMEM_SHARED((ARM, 16), jnp.float32),
            pltpu.VMEM_SHARED((AR1, 16), jnp.float32),
            pltpu.VMEM_SHARED((AR1, 16), jnp.float32),
            pltpu.SemaphoreType.DMA,
        ],
        compiler_params=pltpu.CompilerParams(use_tc_tiling_on_sc=False),
    )
    def k(d0_hbm, d1_hbm, d2_hbm, ones_hbm, z_hbm, o0_hbm, o1_hbm, o2_hbm,
          idx0, idx1, idx2, ones, zb, accM, accP, accQ, sem):
        c = lax.axis_index("c")
        s = lax.axis_index("s")
        w = c * NS + s
        pltpu.sync_copy(d0_hbm.at[pl.ds(w * NCH0, NCH0)], idx0)
        pltpu.sync_copy(d1_hbm.at[pl.ds(w * NCH1, NCH1)], idx1)
        pltpu.sync_copy(d2_hbm.at[pl.ds(w * NCH2, NCH2)], idx2)
        pltpu.sync_copy(ones_hbm, ones)
        pltpu.sync_copy(z_hbm, zb)

        def zcopy(i, carry):
            pltpu.sync_copy(zb, accM.at[pl.ds(s * ART0 + i * ZR, ZR)])
            return carry
        lax.fori_loop(0, ART0 // ZR, zcopy, 0)
        pltpu.sync_copy(zb, accP.at[pl.ds(s * ART1, ART1)])
        pltpu.sync_copy(zb, accQ.at[pl.ds(s * ART1, ART1)])
        plsc.subcore_barrier()

        def chunk0(j, cc):
            pltpu.sync_copy(ones.at[pl.ds(0, CH0)], accM.at[idx0.at[j]], add=True)
            return cc
        lax.fori_loop(0, NCH0, chunk0, 0)

        def chunk1(j, cc):
            pltpu.sync_copy(ones.at[pl.ds(0, CH1)], accP.at[idx1.at[j]], add=True)
            return cc
        lax.fori_loop(0, NCH1, chunk1, 0)

        def chunk2(j, cc):
            pltpu.sync_copy(ones.at[pl.ds(0, CH2)], accQ.at[idx2.at[j]], add=True)
            return cc
        lax.fori_loop(0, NCH2, chunk2, 0)
        plsc.subcore_barrier()

        pltpu.sync_copy(accM.at[pl.ds(s * ART0, ART0)],
                        o0_hbm.at[pl.ds(c * ARM + s * ART0, ART0)])
        pltpu.sync_copy(accP.at[pl.ds(s * ART1, ART1)],
                        o1_hbm.at[pl.ds(c * AR1 + s * ART1, ART1)])
        pltpu.sync_copy(accQ.at[pl.ds(s * ART1, ART1)],
                        o2_hbm.at[pl.ds(c * AR1 + s * ART1, ART1)])

    def run(d0, d1, d2):
        return k(d0, d1, d2, jnp.ones((128, 16), jnp.float32),
                 jnp.zeros((ZR, 16), jnp.float32))
    return run


_rowsum_main = _make_rowsum(E, 80, ARM, 25, 4, 2)
_rowsum_pool = _make_rowsum(POOL_PAD, 40, AR1, 8)
_rowsum_np1 = _make_rowsum(E1_PAD, 64, AR1, 8)
_hist_all = _make_hist_all(100, 40, 64)


def _tc(body, out_shape):
    return pl.pallas_call(body, out_shape=out_shape)


def _halves(o_ref, AR, NV):
    return o_ref[0][:NV] + o_ref[1][:NV]


def _prep0(x, W0, degp):
    def body(x_ref, w_ref, degp_ref, hs_ref, dinv_ref):
        deg = (degp_ref[0] + degp_ref[1])[:N]
        dinv = jnp.where(deg > 0, lax.rsqrt(deg), 0.0)
        h = jnp.dot(x_ref[...], w_ref[...], preferred_element_type=jnp.float32)
        hs_ref[...] = h * dinv[:, 0:1]
        dinv_ref[...] = dinv
    return _tc(body, (jax.ShapeDtypeStruct((N, H), jnp.float32),
                      jax.ShapeDtypeStruct((N, 16), jnp.float32)))(
        x, W0, degp.reshape(2, ARM, 16))


def _make_mid(AR, NV, relu, with_w):
    def body(*refs):
        if with_w:
            acch, dinv, b, g, be, w_ref, out = refs
        else:
            acch, dinv, b, g, be, out = refs
        a = _halves(acch, AR, NV)
        t = a * dinv[:, 0:1] + b[...]
        mean = jnp.mean(t, axis=0, keepdims=True)
        var = jnp.mean((t - mean) ** 2, axis=0, keepdims=True)
        y = (t - mean) * lax.rsqrt(var + EPS) * g[...] + be[...]
        if relu:
            y = jnp.maximum(y, 0.0)
        if with_w:
            out[...] = jnp.dot(y, w_ref[...],
                               preferred_element_type=jnp.float32) * dinv[:, 0:1]
        else:
            out[...] = y

    def run(acch, dinv, b, g, be, W=None):
        args = [acch.reshape(2, AR, 128), dinv,
                b.reshape(1, 128), g.reshape(1, 128), be.reshape(1, 128)]
        if with_w:
            args.append(W)
        return _tc(body, jax.ShapeDtypeStruct((NV, 128), jnp.float32))(*args)
    return run


_mid_main_relu = _make_mid(ARM, N, True, True)
_mid_main_plain = _make_mid(ARM, N, False, False)
_mid_np1_relu = _make_mid(AR1, NP1, True, True)


def _pool_cls(acch, cntp, degp1, xp1, Wl0, bl0, W2):
    def body(acch_ref, cntp_ref, degp_ref, xp_ref, wl_ref, bl_ref, w2_ref,
             x0_ref, hs2_ref, dinv_ref):
        ssum = _halves(acch_ref, AR1, NP1)
        cnt = (cntp_ref[0] + cntp_ref[1])[:NP1, 0:1]
        p = ssum / jnp.maximum(cnt, 1.0)
        logits = jnp.dot(p, wl_ref[...], preferred_element_type=jnp.float32) + bl_ref[...]
        m = jnp.max(logits, axis=1, keepdims=True)
        ex = jnp.exp(logits - m)
        x0 = ex / jnp.sum(ex, axis=1, keepdims=True)
        deg1 = (degp_ref[0] + degp_ref[1])[:NP1]
        dinv1 = jnp.where(deg1 > 0, lax.rsqrt(deg1), 0.0)
        h = (jnp.dot(x0, w2_ref[0:16, :], preferred_element_type=jnp.float32)
             + xp_ref[...] * w2_ref[16:17, :])
        x0_ref[...] = x0
        hs2_ref[...] = h * dinv1[:, 0:1]
        dinv_ref[...] = dinv1
    return _tc(body, (jax.ShapeDtypeStruct((NP1, C), jnp.float32),
                      jax.ShapeDtypeStruct((NP1, H), jnp.float32),
                      jax.ShapeDtypeStruct((NP1, 16), jnp.float32)))(
        acch.reshape(2, AR1, 128), cntp.reshape(2, AR1, 16),
        degp1.reshape(2, AR1, 16), xp1, Wl0, bl0.reshape(1, C), W2)


def _final(acch, dinv1, b3, g3, be3, Wl1, bl1):
    def body(acch_ref, dinv_ref, b_ref, g_ref, be_ref, wl_ref, bl_ref, out_ref):
        t = _halves(acch_ref, AR1, NP1) * dinv_ref[:, 0:1] + b_ref[...]
        mean = jnp.mean(t, axis=0, keepdims=True)
        var = jnp.mean((t - mean) ** 2, axis=0, keepdims=True)
        y = (t - mean) * lax.rsqrt(var + EPS) * g_ref[...] + be_ref[...]
        gm = jnp.mean(y, axis=0, keepdims=True)
        logits = jnp.dot(gm, wl_ref[...], preferred_element_type=jnp.float32) + bl_ref[...]
        m = jnp.max(logits, axis=1, keepdims=True)
        ex = jnp.exp(logits - m)
        out_ref[...] = ex / jnp.sum(ex, axis=1, keepdims=True)
    return _tc(body, jax.ShapeDtypeStruct((1, C), jnp.float32))(
        acch.reshape(2, AR1, 128), dinv1, b3.reshape(1, 128),
        g3.reshape(1, 128), be3.reshape(1, 128), Wl1, bl1.reshape(1, C))


def kernel(x, edge_index, pool1, x_pool1, edge_index_pool1,
           W0, b0, g0, be0, W1, b1, g1, be1, Wl0, bl0,
           W2, b2, g2, be2, W3, b3, g3, be3, Wl1, bl1):
    src0 = edge_index[0].reshape(E // 80, 80)
    dst0 = edge_index[1].reshape(E // 80, 80)
    pool_src = jnp.concatenate(
        [jnp.arange(N, dtype=jnp.int32),
         jnp.zeros((POOL_PAD - N,), jnp.int32)]).reshape(POOL_PAD // 40, 40)
    pool_dst = jnp.concatenate(
        [pool1, jnp.full((POOL_PAD - N,), NP1, jnp.int32)]).reshape(POOL_PAD // 40, 40)
    src1 = jnp.concatenate(
        [edge_index_pool1[0],
         jnp.zeros((E1_PAD - E1,), jnp.int32)]).reshape(E1_PAD // 64, 64)
    dst1 = jnp.concatenate(
        [edge_index_pool1[1],
         jnp.full((E1_PAD - E1,), NP1, jnp.int32)]).reshape(E1_PAD // 64, 64)

    degp0, cntp, degp1 = _hist_all(
        edge_index[1].reshape(E // 100, 100), pool_dst, dst1)

    hs0, dinv0 = _prep0(x, W0, degp0)
    accA = _rowsum_main(src0, dst0, hs0)
    hs1 = _mid_main_relu(accA, dinv0, b0, g0, be0, W1)
    accB = _rowsum_main(src0, dst0, hs1)
    yB = _mid_main_plain(accB, dinv0, b1, g1, be1)

    accP = _rowsum_pool(pool_src, pool_dst, yB)
    x0, hs2, dinv1 = _pool_cls(accP, cntp, degp1, x_pool1, Wl0, bl0, W2)

    accC = _rowsum_np1(src1, dst1, hs2)
    hs3 = _mid_np1_relu(accC, dinv1, b2, g2, be2, W3)
    accD = _rowsum_np1(src1, dst1, hs3)
    out = _final(accD, dinv1, b3, g3, be3, Wl1, bl1)
    return (x0, out)


# speedup vs baseline: 1.0156x; 1.0156x over previous
import functools

import jax
import jax.numpy as jnp
from jax import lax
from jax.experimental import pallas as pl
from jax.experimental.pallas import tpu as pltpu
from jax.experimental.pallas import tpu_sc as plsc

N = 10000
E = 320000
D = 128
H = 128
C = 16
NP1 = 1000
E1 = 16000
EPS = 1e-5

NC = 2
NS = 16
NW = NC * NS

POOL_PAD = 10240
E1_PAD = 16384
ARM = 10240
AR1 = 1024
G = 8


def _zdiv(n, cap):
    for z in range(min(n, cap), 0, -1):
        if n % z == 0:
            return z
    return 1


NBUF = 6
LOOKA = 3


def _make_rowsum(EP, CH, AR, GG, NB=NBUF, LA=LOOKA):
    EPT = EP // NW
    NCH = EPT // CH
    NG = NCH // GG
    ART = AR // NS
    ZR = _zdiv(ART, 32)
    NZ = ART // ZR
    mesh = plsc.VectorSubcoreMesh(core_axis_name="c", subcore_axis_name="s")

    @functools.partial(
        pl.kernel,
        out_type=jax.ShapeDtypeStruct((NC * AR, 128), jnp.float32),
        mesh=mesh,
        scratch_types=[
            pltpu.VMEM((GG, CH), jnp.int32),
            pltpu.VMEM((GG, CH), jnp.int32),
            pltpu.VMEM((NB, CH, 128), jnp.float32),
            pltpu.VMEM((ZR, 128), jnp.float32),
            pltpu.VMEM_SHARED((AR, 128), jnp.float32),
        ] + [pltpu.SemaphoreType.DMA] * (2 * NB),
        compiler_params=pltpu.CompilerParams(use_tc_tiling_on_sc=False),
    )
    def k(src_hbm, dst_hbm, table_hbm, z_hbm, out_hbm, idx_s, idx_d,
          rows, zb, acc, *sems):
        gsem = sems[:NB]
        ssem = sems[NB:]
        c = lax.axis_index("c")
        s = lax.axis_index("s")
        w = c * NS + s
        pltpu.sync_copy(z_hbm, zb)

        def zcopy(i, carry):
            pltpu.sync_copy(zb, acc.at[pl.ds(s * ART + i * ZR, ZR)])
            return carry
        lax.fori_loop(0, NZ, zcopy, 0)
        plsc.subcore_barrier()

        def group(gi, carry):
            base = w * NCH + gi * GG
            pltpu.sync_copy(src_hbm.at[pl.ds(base, GG)], idx_s)
            pltpu.sync_copy(dst_hbm.at[pl.ds(base, GG)], idx_d)

            gh = [None] * NB
            sh = [None] * NB
            for j in range(LA):
                gh[j] = pltpu.async_copy(
                    table_hbm.at[idx_s.at[j]], rows.at[j], gsem[j])
            for j in range(GG):
                b = j % NB
                gh[b].wait()
                sh[b] = pltpu.async_copy(
                    rows.at[b], acc.at[idx_d.at[j]], ssem[b], add=True)
                jn = j + LA
                if jn < GG:
                    bn = jn % NB
                    if sh[bn] is not None:
                        sh[bn].wait()
                        sh[bn] = None
                    gh[bn] = pltpu.async_copy(
                        table_hbm.at[idx_s.at[jn]], rows.at[bn], gsem[bn])
            for b in range(NB):
                if sh[b] is not None:
                    sh[b].wait()
            return carry
        lax.fori_loop(0, NG, group, 0)
        plsc.subcore_barrier()

        pltpu.sync_copy(acc.at[pl.ds(s * ART, ART)],
                        out_hbm.at[pl.ds(c * AR + s * ART, ART)])

    def run(src2d, dst2d, table):
        return k(src2d, dst2d, table, jnp.zeros((ZR, 128), jnp.float32))
    return run


def _make_hist_all(CH0, CH1, CH2):
    NCH0 = E // NW // CH0
    NG0 = NCH0 // G
    NCH1 = POOL_PAD // NW // CH1
    NCH2 = E1_PAD // NW // CH2
    ART0 = ARM // NS
    ART1 = AR1 // NS
    ZR = 64
    mesh = plsc.VectorSubcoreMesh(core_axis_name="c", subcore_axis_name="s")
    S16 = lambda AR: jax.ShapeDtypeStruct((NC * AR, 16), jnp.float32)

    @functools.partial(
        pl.kernel,
        out_type=(S16(ARM), S16(AR1), S16(AR1)),
        mesh=mesh,
        scratch_types=[
            pltpu.VMEM((NCH0, CH0), jnp.int32),
            pltpu.VMEM((NCH1, CH1), jnp.int32),
            pltpu.VMEM((NCH2, CH2), jnp.int32),
            pltpu.VMEM((256, 16), jnp.float32),
            pltpu.VMEM((ZR, 16), jnp.float32),
            pltpu.VMEM_SHARED((ARM, 16), jnp.float32),
            pltpu.VMEM_SHARED((AR1, 16), jnp.float32),
            pltpu.VMEM_SHARED((AR1, 16), jnp.float32),
            pltpu.SemaphoreType.DMA,
        ],
        compiler_params=pltpu.CompilerParams(use_tc_tiling_on_sc=False),
    )
    def k(d0_hbm, d1_hbm, d2_hbm, ones_hbm, z_hbm, o0_hbm, o1_hbm, o2_hbm,
          idx0, idx1, idx2, ones, zb, accM, accP, accQ, sem):
        c = lax.axis_index("c")
        s = lax.axis_index("s")
        w = c * NS + s
        pltpu.sync_copy(d0_hbm.at[pl.ds(w * NCH0, NCH0)], idx0)
        pltpu.sync_copy(d1_hbm.at[pl.ds(w * NCH1, NCH1)], idx1)
        pltpu.sync_copy(d2_hbm.at[pl.ds(w * NCH2, NCH2)], idx2)
        pltpu.sync_copy(ones_hbm, ones)
        pltpu.sync_copy(z_hbm, zb)

        def zcopy(i, carry):
            pltpu.sync_copy(zb, accM.at[pl.ds(s * ART0 + i * ZR, ZR)])
            return carry
        lax.fori_loop(0, ART0 // ZR, zcopy, 0)
        pltpu.sync_copy(zb, accP.at[pl.ds(s * ART1, ART1)])
        pltpu.sync_copy(zb, accQ.at[pl.ds(s * ART1, ART1)])
        plsc.subcore_barrier()

        def chunk0(j, cc):
            pltpu.sync_copy(ones.at[pl.ds(0, CH0)], accM.at[idx0.at[j]], add=True)
            return cc
        lax.fori_loop(0, NCH0, chunk0, 0)

        def chunk1(j, cc):
            pltpu.sync_copy(ones.at[pl.ds(0, CH1)], accP.at[idx1.at[j]], add=True)
            return cc
        lax.fori_loop(0, NCH1, chunk1, 0)

        def chunk2(j, cc):
            pltpu.sync_copy(ones.at[pl.ds(0, CH2)], accQ.at[idx2.at[j]], add=True)
            return cc
        lax.fori_loop(0, NCH2, chunk2, 0)
        plsc.subcore_barrier()

        pltpu.sync_copy(accM.at[pl.ds(s * ART0, ART0)],
                        o0_hbm.at[pl.ds(c * ARM + s * ART0, ART0)])
        pltpu.sync_copy(accP.at[pl.ds(s * ART1, ART1)],
                        o1_hbm.at[pl.ds(c * AR1 + s * ART1, ART1)])
        pltpu.sync_copy(accQ.at[pl.ds(s * ART1, ART1)],
                        o2_hbm.at[pl.ds(c * AR1 + s * ART1, ART1)])

    def run(d0, d1, d2):
        return k(d0, d1, d2, jnp.ones((256, 16), jnp.float32),
                 jnp.zeros((ZR, 16), jnp.float32))
    return run


_rowsum_main = _make_rowsum(E, 50, ARM, 40)
_rowsum_pool = _make_rowsum(POOL_PAD, 80, AR1, 4)
_rowsum_np1 = _make_rowsum(E1_PAD, 128, AR1, 4)
_hist_all = _make_hist_all(200, 80, 128)


def _tc(body, out_shape):
    return pl.pallas_call(body, out_shape=out_shape)


def _halves(o_ref, AR, NV):
    return o_ref[0][:NV] + o_ref[1][:NV]


def _prep0(x, W0, degp):
    def body(x_ref, w_ref, degp_ref, hs_ref, dinv_ref):
        deg = (degp_ref[0] + degp_ref[1])[:N]
        dinv = jnp.where(deg > 0, lax.rsqrt(deg), 0.0)
        h = jnp.dot(x_ref[...], w_ref[...], preferred_element_type=jnp.float32)
        hs_ref[...] = h * dinv[:, 0:1]
        dinv_ref[...] = dinv
    return _tc(body, (jax.ShapeDtypeStruct((N, H), jnp.float32),
                      jax.ShapeDtypeStruct((N, 16), jnp.float32)))(
        x, W0, degp.reshape(2, ARM, 16))


def _make_mid(AR, NV, relu, with_w):
    def body(*refs):
        if with_w:
            acch, dinv, b, g, be, w_ref, out = refs
        else:
            acch, dinv, b, g, be, out = refs
        a = _halves(acch, AR, NV)
        t = a * dinv[:, 0:1] + b[...]
        mean = jnp.mean(t, axis=0, keepdims=True)
        var = jnp.mean((t - mean) ** 2, axis=0, keepdims=True)
        y = (t - mean) * lax.rsqrt(var + EPS) * g[...] + be[...]
        if relu:
            y = jnp.maximum(y, 0.0)
        if with_w:
            out[...] = jnp.dot(y, w_ref[...],
                               preferred_element_type=jnp.float32) * dinv[:, 0:1]
        else:
            out[...] = y

    def run(acch, dinv, b, g, be, W=None):
        args = [acch.reshape(2, AR, 128), dinv,
                b.reshape(1, 128), g.reshape(1, 128), be.reshape(1, 128)]
        if with_w:
            args.append(W)
        return _tc(body, jax.ShapeDtypeStruct((NV, 128), jnp.float32))(*args)
    return run


_mid_main_relu = _make_mid(ARM, N, True, True)
_mid_main_plain = _make_mid(ARM, N, False, False)
_mid_np1_relu = _make_mid(AR1, NP1, True, True)


def _pool_cls(acch, cntp, degp1, xp1, Wl0, bl0, W2):
    def body(acch_ref, cntp_ref, degp_ref, xp_ref, wl_ref, bl_ref, w2_ref,
             x0_ref, hs2_ref, dinv_ref):
        ssum = _halves(acch_ref, AR1, NP1)
        cnt = (cntp_ref[0] + cntp_ref[1])[:NP1, 0:1]
        p = ssum / jnp.maximum(cnt, 1.0)
        logits = jnp.dot(p, wl_ref[...], preferred_element_type=jnp.float32) + bl_ref[...]
        m = jnp.max(logits, axis=1, keepdims=True)
        ex = jnp.exp(logits - m)
        x0 = ex / jnp.sum(ex, axis=1, keepdims=True)
        deg1 = (degp_ref[0] + degp_ref[1])[:NP1]
        dinv1 = jnp.where(deg1 > 0, lax.rsqrt(deg1), 0.0)
        h = (jnp.dot(x0, w2_ref[0:16, :], preferred_element_type=jnp.float32)
             + xp_ref[...] * w2_ref[16:17, :])
        x0_ref[...] = x0
        hs2_ref[...] = h * dinv1[:, 0:1]
        dinv_ref[...] = dinv1
    return _tc(body, (jax.ShapeDtypeStruct((NP1, C), jnp.float32),
                      jax.ShapeDtypeStruct((NP1, H), jnp.float32),
                      jax.ShapeDtypeStruct((NP1, 16), jnp.float32)))(
        acch.reshape(2, AR1, 128), cntp.reshape(2, AR1, 16),
        degp1.reshape(2, AR1, 16), xp1, Wl0, bl0.reshape(1, C), W2)


def _final(acch, dinv1, b3, g3, be3, Wl1, bl1):
    def body(acch_ref, dinv_ref, b_ref, g_ref, be_ref, wl_ref, bl_ref, out_ref):
        t = _halves(acch_ref, AR1, NP1) * dinv_ref[:, 0:1] + b_ref[...]
        mean = jnp.mean(t, axis=0, keepdims=True)
        var = jnp.mean((t - mean) ** 2, axis=0, keepdims=True)
        y = (t - mean) * lax.rsqrt(var + EPS) * g_ref[...] + be_ref[...]
        gm = jnp.mean(y, axis=0, keepdims=True)
        logits = jnp.dot(gm, wl_ref[...], preferred_element_type=jnp.float32) + bl_ref[...]
        m = jnp.max(logits, axis=1, keepdims=True)
        ex = jnp.exp(logits - m)
        out_ref[...] = ex / jnp.sum(ex, axis=1, keepdims=True)
    return _tc(body, jax.ShapeDtypeStruct((1, C), jnp.float32))(
        acch.reshape(2, AR1, 128), dinv1, b3.reshape(1, 128),
        g3.reshape(1, 128), be3.reshape(1, 128), Wl1, bl1.reshape(1, C))


def kernel(x, edge_index, pool1, x_pool1, edge_index_pool1,
           W0, b0, g0, be0, W1, b1, g1, be1, Wl0, bl0,
           W2, b2, g2, be2, W3, b3, g3, be3, Wl1, bl1):
    src0 = edge_index[0].reshape(E // 50, 50)
    dst0 = edge_index[1].reshape(E // 50, 50)
    pool_src = jnp.concatenate(
        [jnp.arange(N, dtype=jnp.int32),
         jnp.zeros((POOL_PAD - N,), jnp.int32)]).reshape(POOL_PAD // 80, 80)
    pool_dst = jnp.concatenate(
        [pool1, jnp.full((POOL_PAD - N,), NP1, jnp.int32)]).reshape(POOL_PAD // 80, 80)
    src1 = jnp.concatenate(
        [edge_index_pool1[0],
         jnp.zeros((E1_PAD - E1,), jnp.int32)]).reshape(E1_PAD // 128, 128)
    dst1 = jnp.concatenate(
        [edge_index_pool1[1],
         jnp.full((E1_PAD - E1,), NP1, jnp.int32)]).reshape(E1_PAD // 128, 128)

    degp0, cntp, degp1 = _hist_all(
        edge_index[1].reshape(E // 200, 200), pool_dst, dst1)

    hs0, dinv0 = _prep0(x, W0, degp0)
    accA = _rowsum_main(src0, dst0, hs0)
    hs1 = _mid_main_relu(accA, dinv0, b0, g0, be0, W1)
    accB = _rowsum_main(src0, dst0, hs1)
    yB = _mid_main_plain(accB, dinv0, b1, g1, be1)

    accP = _rowsum_pool(pool_src, pool_dst, yB)
    x0, hs2, dinv1 = _pool_cls(accP, cntp, degp1, x_pool1, Wl0, bl0, W2)

    accC = _rowsum_np1(src1, dst1, hs2)
    hs3 = _mid_np1_relu(accC, dinv1, b2, g2, be2, W3)
    accD = _rowsum_np1(src1, dst1, hs3)
    out = _final(accD, dinv1, b3, g3, be3, Wl1, bl1)
    return (x0, out)
